# baseline (speedup 1.0000x reference)
import functools
import math

import jax
import jax.numpy as jnp
from jax.experimental import pallas as pl
from jax.experimental.pallas import tpu as pltpu


def _mm_kernel(x_ref, w_ref, b_ref, o_ref, *, relu):
    acc = jnp.dot(x_ref[...], w_ref[...], preferred_element_type=jnp.float32)
    acc = acc + b_ref[...]
    if relu:
        acc = jnp.maximum(acc, 0.0)
    o_ref[...] = acc.astype(o_ref.dtype)


def _matmul_bias(x, w, b, *, relu, out_dtype):
    M, K = x.shape
    N = w.shape[1]
    TM = M if M <= 16384 else 8192
    TN = 512 if (N % 512 == 0 and N > 512) else N
    grid = (pl.cdiv(M, TM), N // TN)
    return pl.pallas_call(
        functools.partial(_mm_kernel, relu=relu),
        out_shape=jax.ShapeDtypeStruct((M, N), out_dtype),
        grid=grid,
        in_specs=[
            pl.BlockSpec((TM, K), lambda i, j: (i, 0)),
            pl.BlockSpec((K, TN), lambda i, j: (0, j)),
            pl.BlockSpec((1, TN), lambda i, j: (0, j)),
        ],
        out_specs=pl.BlockSpec((TM, TN), lambda i, j: (i, j)),
        compiler_params=pltpu.CompilerParams(
            dimension_semantics=("parallel", "parallel"),
            vmem_limit_bytes=64 * 1024 * 1024),
    )(x, w, b)


def _conv_tap_kernel(x_ref, w_ref, b_ref, o_ref, *, k, tm):
    parts = []
    for dj in range(k):
        xs = x_ref[dj:tm, :]
        if dj:
            xs = jnp.pad(xs, ((0, dj), (0, 0)))
        parts.append(xs)
    xcat = jnp.concatenate(parts, axis=1)
    acc = jnp.dot(xcat, w_ref[...], preferred_element_type=jnp.float32)
    acc = jnp.maximum(acc + b_ref[...], 0.0)
    o_ref[...] = acc.astype(o_ref.dtype)


def _conv_layer_tap(a, w, bias, k, cin, cout):
    B, H, W, _ = a.shape
    Ho, Wo = H - k + 1, W - k + 1
    kcin = k * cin
    kcin_p = ((kcin + 127) // 128) * 128

    aH = jnp.concatenate([a[:, di:di + Ho] for di in range(k)], axis=-1)
    if kcin_p != kcin:
        aH = jnp.pad(aH, ((0, 0), (0, 0), (0, 0), (0, kcin_p - kcin)))
    X = aH.reshape(B * Ho * W, kcin_p)

    wr = w.reshape(k, k, cin, -1).transpose(1, 0, 2, 3).reshape(k, kcin, -1)
    wr = jnp.pad(wr, ((0, 0), (0, kcin_p - kcin), (0, 0)))
    wr = wr.reshape(k * kcin_p, -1)[:, :cout].astype(jnp.bfloat16)
    br = bias[:, :cout]

    panels = B * Ho
    P = 16
    while panels % P:
        P //= 2
    TM = P * W
    out = pl.pallas_call(
        functools.partial(_conv_tap_kernel, k=k, tm=TM),
        out_shape=jax.ShapeDtypeStruct((panels * W, cout), jnp.bfloat16),
        grid=(panels // P,),
        in_specs=[
            pl.BlockSpec((TM, kcin_p), lambda i: (i, 0)),
            pl.BlockSpec((k * kcin_p, cout), lambda i: (0, 0)),
            pl.BlockSpec((1, cout), lambda i: (0, 0)),
        ],
        out_specs=pl.BlockSpec((TM, cout), lambda i: (i, 0)),
        compiler_params=pltpu.CompilerParams(
            dimension_semantics=("parallel",),
            vmem_limit_bytes=64 * 1024 * 1024),
    )(X, wr, br)
    return out.reshape(B, Ho, W, cout)[:, :, :Wo, :]


def _conv0_im2col(x4, w, bias, k, cout):
    B, H, W, _ = x4.shape
    Ho, Wo = H - k + 1, W - k + 1
    patches = jnp.concatenate(
        [x4[:, di:di + Ho, dj:dj + Wo, :] for di in range(k) for dj in range(k)],
        axis=-1)
    pm = patches.reshape(B * Ho * Wo, k * k)
    out = _matmul_bias(pm, w[:, :cout].astype(jnp.bfloat16), bias[:, :cout],
                       relu=True, out_dtype=jnp.bfloat16)
    return out.reshape(B, Ho, Wo, cout)


def _pool2(x):
    B, H, W, C = x.shape
    x = x[:, :H // 2 * 2, :W // 2 * 2, :].reshape(B, H // 2, 2, W // 2, 2, C)
    return x.max(axis=(2, 4))


def _bilstm_kernel(gx_ref, whh_ref, len_ref, y_ref, *, T, B, H):
    d = pl.program_id(0)
    lens = len_ref[...]
    whh = whh_ref[...]

    def sweep(order):
        h = jnp.zeros((B, H), jnp.float32)
        c = jnp.zeros((B, H), jnp.float32)
        for t in order:
            g = gx_ref[t].astype(jnp.float32) + jnp.dot(
                h.astype(jnp.bfloat16), whh, preferred_element_type=jnp.float32)
            i_g = jax.nn.sigmoid(g[:, :H])
            f_g = jax.nn.sigmoid(g[:, H:2 * H])
            g_g = jnp.tanh(g[:, 2 * H:3 * H])
            o_g = jax.nn.sigmoid(g[:, 3 * H:])
            c_n = f_g * c + i_g * g_g
            h_n = o_g * jnp.tanh(c_n)
            live = t < lens
            h = jnp.where(live, h_n, h)
            c = jnp.where(live, c_n, c)
            y_ref[t] = jnp.where(live, h_n, 0.0).astype(y_ref.dtype)

    @pl.when(d == 0)
    def _():
        sweep(tuple(range(T)))

    @pl.when(d == 1)
    def _():
        sweep(tuple(range(T - 1, -1, -1)))


def _bilstm_layer(seq, lens_b1, wih, b, whh):
    T, B, D = seq.shape
    H = whh.shape[1]
    gx = _matmul_bias(seq.reshape(T * B, D), wih.astype(jnp.bfloat16), b,
                      relu=False, out_dtype=jnp.bfloat16).reshape(T, B, 8 * H)
    y2 = pl.pallas_call(
        functools.partial(_bilstm_kernel, T=T, B=B, H=H),
        out_shape=jax.ShapeDtypeStruct((2, T, B, H), jnp.bfloat16),
        grid=(2,),
        in_specs=[
            pl.BlockSpec((T, B, 4 * H), lambda d: (0, 0, d)),
            pl.BlockSpec((None, H, 4 * H), lambda d: (d, 0, 0)),
            pl.BlockSpec((B, 1), lambda d: (0, 0)),
        ],
        out_specs=pl.BlockSpec((None, T, B, H), lambda d: (d, 0, 0, 0)),
        compiler_params=pltpu.CompilerParams(
            dimension_semantics=("parallel",)),
    )(gx, whh, lens_b1)
    return jnp.concatenate([y2[0], y2[1]], axis=-1)


def kernel(conv0_w, conv0_bias, conv1_w, conv1_bias, conv2_w, conv2_bias,
           conv3_w, conv3_bias, lstm0_wih, lstm0_b, lstm0_whh, lstm1_wih,
           lstm1_b, lstm1_whh, lstm2_wih, lstm2_b, lstm2_whh, fc_w, fc_b,
           x, input_lengths):
    out_lengths = input_lengths
    for k in (5, 3, 3, 5):
        out_lengths = out_lengths - k + 1
        out_lengths = (out_lengths - 2) // 2 + 1

    h = x[..., None].astype(jnp.bfloat16)
    h = _pool2(_conv0_im2col(h, conv0_w, conv0_bias, 5, 32))
    h = _pool2(_conv_layer_tap(h, conv1_w, conv1_bias, 3, 32, 32))
    h = _pool2(_conv_layer_tap(h, conv2_w, conv2_bias, 3, 32, 64))
    h = _pool2(_conv_layer_tap(h, conv3_w, conv3_bias, 5, 64, 64))

    B, Hc, Wt, C = h.shape
    seq = jnp.transpose(h, (2, 0, 1, 3)).reshape(Wt, B, Hc * C)

    lens_b1 = out_lengths.reshape(B, 1).astype(jnp.int32)
    seq = _bilstm_layer(seq, lens_b1, lstm0_wih, lstm0_b, lstm0_whh)
    seq = _bilstm_layer(seq, lens_b1, lstm1_wih, lstm1_b, lstm1_whh)
    seq = _bilstm_layer(seq, lens_b1, lstm2_wih, lstm2_b, lstm2_whh)

    T2, _, D2 = seq.shape
    tgt = fc_w.shape[1]
    tgt_p = ((tgt + 127) // 128) * 128
    fw = jnp.pad(fc_w, ((0, 0), (0, tgt_p - tgt))).astype(jnp.bfloat16)
    fb = jnp.pad(fc_b, ((0, 0), (0, tgt_p - tgt)))
    y = _matmul_bias(seq.reshape(T2 * B, D2), fw, fb,
                     relu=False, out_dtype=jnp.float32)[:, :tgt]
    y = jnp.transpose(y.reshape(T2, B, tgt), (1, 0, 2))
    return y, out_lengths


# speedup vs baseline: 1.2212x; 1.2212x over previous
import functools
import math

import jax
import jax.numpy as jnp
from jax.experimental import pallas as pl
from jax.experimental.pallas import tpu as pltpu


def _mm_kernel(x_ref, w_ref, b_ref, o_ref, *, relu):
    acc = jnp.dot(x_ref[...], w_ref[...], preferred_element_type=jnp.float32)
    acc = acc + b_ref[...]
    if relu:
        acc = jnp.maximum(acc, 0.0)
    o_ref[...] = acc.astype(o_ref.dtype)


def _matmul_bias(x, w, b, *, relu, out_dtype):
    M, K = x.shape
    N = w.shape[1]
    TM = M if M <= 16384 else 8192
    TN = 512 if (N % 512 == 0 and N > 512) else N
    grid = (pl.cdiv(M, TM), N // TN)
    return pl.pallas_call(
        functools.partial(_mm_kernel, relu=relu),
        out_shape=jax.ShapeDtypeStruct((M, N), out_dtype),
        grid=grid,
        in_specs=[
            pl.BlockSpec((TM, K), lambda i, j: (i, 0)),
            pl.BlockSpec((K, TN), lambda i, j: (0, j)),
            pl.BlockSpec((1, TN), lambda i, j: (0, j)),
        ],
        out_specs=pl.BlockSpec((TM, TN), lambda i, j: (i, j)),
        compiler_params=pltpu.CompilerParams(
            dimension_semantics=("parallel", "parallel"),
            vmem_limit_bytes=64 * 1024 * 1024),
    )(x, w, b)


def _conv_tap_kernel(*refs, k, cin, cout, tm):
    x_refs, w_ref, b_ref, o_ref = refs[:k], refs[k], refs[k + 1], refs[-1]
    acc = b_ref[...].astype(jnp.float32) * jnp.ones((tm, cout), jnp.float32)
    for di in range(k):
        for dj in range(k):
            xs = x_refs[di][dj:tm, :]
            if dj:
                xs = jnp.pad(xs, ((0, dj), (0, 0)))
            wt = w_ref[(di * k + dj) * cin:(di * k + dj + 1) * cin, :]
            acc += jnp.dot(xs, wt, preferred_element_type=jnp.float32)
    o_ref[...] = jnp.maximum(acc, 0.0).astype(o_ref.dtype)


def _conv_layer_tap(a, w, bias, k, cin, cout):
    B, H, W, _ = a.shape
    Ho, Wo = H - k + 1, W - k + 1
    xs = [a[:, di:di + Ho].reshape(B * Ho * W, cin) for di in range(k)]
    wr = w[:, :cout].astype(jnp.bfloat16)
    br = bias[:, :cout]

    panels = B * Ho
    P = 16
    while panels % P:
        P //= 2
    TM = P * W
    out = pl.pallas_call(
        functools.partial(_conv_tap_kernel, k=k, cin=cin, cout=cout, tm=TM),
        out_shape=jax.ShapeDtypeStruct((panels * W, cout), jnp.bfloat16),
        grid=(panels // P,),
        in_specs=[pl.BlockSpec((TM, cin), lambda i: (i, 0)) for _ in range(k)]
        + [
            pl.BlockSpec((k * k * cin, cout), lambda i: (0, 0)),
            pl.BlockSpec((1, cout), lambda i: (0, 0)),
        ],
        out_specs=pl.BlockSpec((TM, cout), lambda i: (i, 0)),
        compiler_params=pltpu.CompilerParams(
            dimension_semantics=("parallel",),
            vmem_limit_bytes=64 * 1024 * 1024),
    )(*xs, wr, br)
    return out.reshape(B, Ho, W, cout)[:, :, :Wo, :]


def _mmT_kernel(x_ref, w_ref, b_ref, o_ref):
    acc = jax.lax.dot_general(
        x_ref[...], w_ref[...], (((0,), (0,)), ((), ())),
        preferred_element_type=jnp.float32)
    o_ref[...] = jnp.maximum(acc + b_ref[...], 0.0).astype(o_ref.dtype)


def _conv0_im2col(x4, w, bias, k, cout):
    B, H, W, _ = x4.shape
    Ho, Wo = H - k + 1, W - k + 1
    x3 = x4[..., 0]
    pT = jnp.stack(
        [x3[:, di:di + Ho, dj:dj + Wo] for di in range(k) for dj in range(k)],
        axis=0).reshape(k * k, B * Ho * Wo)
    M = B * Ho * Wo
    TM = 16384
    out = pl.pallas_call(
        _mmT_kernel,
        out_shape=jax.ShapeDtypeStruct((M, cout), jnp.bfloat16),
        grid=(pl.cdiv(M, TM),),
        in_specs=[
            pl.BlockSpec((k * k, TM), lambda i: (0, i)),
            pl.BlockSpec((k * k, cout), lambda i: (0, 0)),
            pl.BlockSpec((1, cout), lambda i: (0, 0)),
        ],
        out_specs=pl.BlockSpec((TM, cout), lambda i: (i, 0)),
        compiler_params=pltpu.CompilerParams(
            dimension_semantics=("parallel",),
            vmem_limit_bytes=64 * 1024 * 1024),
    )(pT, w[:, :cout].astype(jnp.bfloat16), bias[:, :cout])
    return out.reshape(B, Ho, Wo, cout)


def _pool2(x):
    B, H, W, C = x.shape
    x = x[:, :H // 2 * 2, :W // 2 * 2, :].reshape(B, H // 2, 2, W // 2, 2, C)
    return x.max(axis=(2, 4))


def _bilstm_kernel(gx_ref, whh_ref, len_ref, y_ref, *, T, B, H):
    d = pl.program_id(0)
    lens = len_ref[...]
    whh = whh_ref[...]

    def sweep(order):
        h = jnp.zeros((B, H), jnp.float32)
        c = jnp.zeros((B, H), jnp.float32)
        for t in order:
            g = gx_ref[t].astype(jnp.float32) + jnp.dot(
                h.astype(jnp.bfloat16), whh, preferred_element_type=jnp.float32)
            i_g = jax.nn.sigmoid(g[:, :H])
            f_g = jax.nn.sigmoid(g[:, H:2 * H])
            g_g = jnp.tanh(g[:, 2 * H:3 * H])
            o_g = jax.nn.sigmoid(g[:, 3 * H:])
            c_n = f_g * c + i_g * g_g
            h_n = o_g * jnp.tanh(c_n)
            live = t < lens
            h = jnp.where(live, h_n, h)
            c = jnp.where(live, c_n, c)
            y_ref[t] = jnp.where(live, h_n, 0.0).astype(y_ref.dtype)

    @pl.when(d == 0)
    def _():
        sweep(tuple(range(T)))

    @pl.when(d == 1)
    def _():
        sweep(tuple(range(T - 1, -1, -1)))


def _bilstm_layer(seq, lens_b1, wih, b, whh):
    T, B, D = seq.shape
    H = whh.shape[1]
    gx = _matmul_bias(seq.reshape(T * B, D), wih.astype(jnp.bfloat16), b,
                      relu=False, out_dtype=jnp.bfloat16).reshape(T, B, 8 * H)
    y2 = pl.pallas_call(
        functools.partial(_bilstm_kernel, T=T, B=B, H=H),
        out_shape=jax.ShapeDtypeStruct((2, T, B, H), jnp.bfloat16),
        grid=(2,),
        in_specs=[
            pl.BlockSpec((T, B, 4 * H), lambda d: (0, 0, d)),
            pl.BlockSpec((None, H, 4 * H), lambda d: (d, 0, 0)),
            pl.BlockSpec((B, 1), lambda d: (0, 0)),
        ],
        out_specs=pl.BlockSpec((None, T, B, H), lambda d: (d, 0, 0, 0)),
        compiler_params=pltpu.CompilerParams(
            dimension_semantics=("parallel",)),
    )(gx, whh, lens_b1)
    return jnp.concatenate([y2[0], y2[1]], axis=-1)


def kernel(conv0_w, conv0_bias, conv1_w, conv1_bias, conv2_w, conv2_bias,
           conv3_w, conv3_bias, lstm0_wih, lstm0_b, lstm0_whh, lstm1_wih,
           lstm1_b, lstm1_whh, lstm2_wih, lstm2_b, lstm2_whh, fc_w, fc_b,
           x, input_lengths):
    out_lengths = input_lengths
    for k in (5, 3, 3, 5):
        out_lengths = out_lengths - k + 1
        out_lengths = (out_lengths - 2) // 2 + 1

    h = x[..., None].astype(jnp.bfloat16)
    h = _pool2(_conv0_im2col(h, conv0_w, conv0_bias, 5, 32))
    h = _pool2(_conv_layer_tap(h, conv1_w, conv1_bias, 3, 32, 32))
    h = _pool2(_conv_layer_tap(h, conv2_w, conv2_bias, 3, 32, 64))
    h = _pool2(_conv_layer_tap(h, conv3_w, conv3_bias, 5, 64, 64))

    B, Hc, Wt, C = h.shape
    seq = jnp.transpose(h, (2, 0, 1, 3)).reshape(Wt, B, Hc * C)

    lens_b1 = out_lengths.reshape(B, 1).astype(jnp.int32)
    seq = _bilstm_layer(seq, lens_b1, lstm0_wih, lstm0_b, lstm0_whh)
    seq = _bilstm_layer(seq, lens_b1, lstm1_wih, lstm1_b, lstm1_whh)
    seq = _bilstm_layer(seq, lens_b1, lstm2_wih, lstm2_b, lstm2_whh)

    T2, _, D2 = seq.shape
    tgt = fc_w.shape[1]
    tgt_p = ((tgt + 127) // 128) * 128
    fw = jnp.pad(fc_w, ((0, 0), (0, tgt_p - tgt))).astype(jnp.bfloat16)
    fb = jnp.pad(fc_b, ((0, 0), (0, tgt_p - tgt)))
    y = _matmul_bias(seq.reshape(T2 * B, D2), fw, fb,
                     relu=False, out_dtype=jnp.float32)[:, :tgt]
    y = jnp.transpose(y.reshape(T2, B, tgt), (1, 0, 2))
    return y, out_lengths
